# baseline (speedup 1.0000x reference)
import functools

import jax
import jax.numpy as jnp
from jax import lax
from jax.experimental import pallas as pl
from jax.experimental.pallas import tpu as pltpu
from jax.experimental.pallas import tpu_sc as plsc

_LANES = 16
_UNROLL = 16
_TR, _TC = 8, 128


def _xlane_take(x, perm):
    dnums = lax.GatherDimensionNumbers(
        offset_dims=(), collapsed_slice_dims=(0,), start_index_map=(0,))
    return lax.gather(x, perm[:, None], dnums, slice_sizes=(1,),
                      mode=lax.GatherScatterMode.PROMISE_IN_BOUNDS)


def _merge(m_a, i_a, m_b, i_b):
    take_b = (m_b > m_a) | ((m_b == m_a) & (i_b < i_a))
    return jnp.where(take_b, m_b, m_a), jnp.where(take_b, i_b, i_a)


def _make_argmax(rows, cols):
    info = plsc.get_sparse_core_info()
    ncores, nsub = info.num_cores, info.num_subcores
    nworkers = ncores * nsub
    assert rows % nworkers == 0 and rows % _TR == 0 and cols % _TC == 0
    rows_per = rows // nworkers
    assert rows_per <= _LANES
    assert cols % (_LANES * _UNROLL) == 0
    steps = cols // (_LANES * _UNROLL)
    segs = cols // _TC
    seg_per_step = (_LANES * _UNROLL) // _TC

    mesh = plsc.VectorSubcoreMesh(core_axis_name="c", subcore_axis_name="s")

    @functools.partial(
        pl.kernel,
        out_type=jax.ShapeDtypeStruct((rows,), jnp.int32),
        mesh=mesh,
        scratch_types=[
            pltpu.VMEM((segs, _TC), jnp.float32),
            pltpu.VMEM((segs, _TC), jnp.float32),
            pltpu.VMEM((_LANES,), jnp.int32),
            pltpu.VMEM((_LANES,), jnp.int32),
            pltpu.SemaphoreType.DMA,
            pltpu.SemaphoreType.DMA,
            pltpu.SemaphoreType.DMA,
        ],
    )
    def argmax_sc(x_hbm, win_hbm, in0, in1, cols_v, rows_v, sem0, sem1, sems):
        wid = lax.axis_index("c") * nsub + lax.axis_index("s")
        row0 = wid * rows_per

        bufs = (in0, in1)
        sems_in = (sem0, sem1)

        def start_in(r, buf, sem):
            rr = row0 + r
            return pltpu.async_copy(
                x_hbm.at[rr // _TR, :, rr % _TR, :], buf, sem)

        in_copies = [None] * rows_per
        for r in range(min(2, rows_per)):
            in_copies[r] = start_in(r, bufs[r % 2], sems_in[r % 2])

        lane = lax.iota(jnp.int32, _LANES)
        winners = jnp.zeros((_LANES,), jnp.int32)
        neg_inf = jnp.full((_LANES,), -jnp.inf, jnp.float32)
        zero_i = jnp.zeros((_LANES,), jnp.int32)

        for r in range(rows_per):
            in_copies[r].wait()
            buf = bufs[r % 2]

            def step(j, carry):
                ms, tags = carry
                new_ms, new_tags = [], []
                for u in range(_UNROLL):
                    v = buf[j * seg_per_step + u // (_TC // _LANES),
                            pl.ds((u % (_TC // _LANES)) * _LANES, _LANES)]
                    gt = v > ms[u]
                    new_ms.append(jnp.where(gt, v, ms[u]))
                    new_tags.append(jnp.where(gt, j, tags[u]))
                return tuple(new_ms), tuple(new_tags)

            init = ((neg_inf,) * _UNROLL, (zero_i,) * _UNROLL)
            ms, tags = lax.fori_loop(0, steps, step, init)

            pairs = [
                (ms[u], tags[u] * (_UNROLL * _LANES) + (u * _LANES) + lane)
                for u in range(_UNROLL)
            ]
            while len(pairs) > 1:
                nxt = []
                for p in range(0, len(pairs), 2):
                    nxt.append(_merge(*pairs[p], *pairs[p + 1]))
                pairs = nxt
            m, idx = pairs[0]

            for k in (8, 4, 2, 1):
                perm = lane ^ k
                m2 = _xlane_take(m, perm)
                i2 = _xlane_take(idx, perm)
                m, idx = _merge(m, idx, m2, i2)

            if r == 0:
                winners = idx
            else:
                winners = jnp.where(lane == r, idx, winners)

            if r + 2 < rows_per:
                in_copies[r + 2] = start_in(r + 2, bufs[r % 2], sems_in[r % 2])

        cols_v[...] = winners
        rows_v[...] = row0 + jnp.where(lane < rows_per, lane, 0)
        pltpu.async_copy(cols_v, win_hbm.at[rows_v], sems).wait()

    return argmax_sc


def _onehot_tc_body(rows_blk, cols):
    def body(win_ref, out_ref):
        i = pl.program_id(0)
        col_iota = lax.broadcasted_iota(jnp.int32, (1, cols), 1)
        for j in range(rows_blk):
            w = win_ref[i * rows_blk + j]
            out_ref[pl.ds(j, 1), :] = (col_iota == w).astype(jnp.float32)
    return body


def _make_onehot(rows, cols, rows_blk=8):
    return pl.pallas_call(
        _onehot_tc_body(rows_blk, cols),
        grid=(rows // rows_blk,),
        in_specs=[pl.BlockSpec(memory_space=pltpu.SMEM)],
        out_specs=pl.BlockSpec((rows_blk, cols), lambda i: (i, 0)),
        out_shape=jax.ShapeDtypeStruct((rows, cols), jnp.float32),
    )


def kernel(tensor):
    rows, cols = tensor.shape
    x4 = tensor.reshape(rows // _TR, _TR, cols // _TC, _TC).transpose(0, 2, 1, 3)
    winners = _make_argmax(rows, cols)(x4)
    return _make_onehot(rows, cols)(winners)


# speedup vs baseline: 2.9916x; 2.9916x over previous
import functools

import jax
import jax.numpy as jnp
from jax import lax
from jax.experimental import pallas as pl
from jax.experimental.pallas import tpu as pltpu
from jax.experimental.pallas import tpu_sc as plsc

_LANES = 16
_UNROLL = 16
_TR, _TC = 8, 128


def _xlane_take(x, perm):
    dnums = lax.GatherDimensionNumbers(
        offset_dims=(), collapsed_slice_dims=(0,), start_index_map=(0,))
    return lax.gather(x, perm[:, None], dnums, slice_sizes=(1,),
                      mode=lax.GatherScatterMode.PROMISE_IN_BOUNDS)


def _merge(m_a, i_a, m_b, i_b):
    take_b = (m_b > m_a) | ((m_b == m_a) & (i_b < i_a))
    return jnp.where(take_b, m_b, m_a), jnp.where(take_b, i_b, i_a)


def _make_argmax(rows, cols):
    info = plsc.get_sparse_core_info()
    ncores, nsub = info.num_cores, info.num_subcores
    nworkers = ncores * nsub
    assert rows % nworkers == 0 and rows % _TR == 0 and cols % _TC == 0
    rows_per = rows // nworkers
    assert rows_per <= _LANES
    assert cols % (_LANES * _UNROLL) == 0
    steps = cols // (_LANES * _UNROLL)
    segs = cols // _TC
    seg_per_step = (_LANES * _UNROLL) // _TC

    mesh = plsc.VectorSubcoreMesh(core_axis_name="c", subcore_axis_name="s")

    @functools.partial(
        pl.kernel,
        out_type=jax.ShapeDtypeStruct((nworkers, _LANES), jnp.int32),
        mesh=mesh,
        scratch_types=[
            pltpu.VMEM((segs, _TC), jnp.float32),
            pltpu.VMEM((segs, _TC), jnp.float32),
            pltpu.VMEM((_LANES,), jnp.int32),
            pltpu.SemaphoreType.DMA,
            pltpu.SemaphoreType.DMA,
        ],
    )
    def argmax_sc(x_hbm, win_hbm, in0, in1, cols_v, sem0, sem1):
        wid = lax.axis_index("c") * nsub + lax.axis_index("s")
        row0 = wid * rows_per

        bufs = (in0, in1)
        sems_in = (sem0, sem1)

        def start_in(r, buf, sem):
            rr = row0 + r
            return pltpu.async_copy(
                x_hbm.at[rr // _TR, :, rr % _TR, :], buf, sem)

        in_copies = [None] * rows_per
        for r in range(min(2, rows_per)):
            in_copies[r] = start_in(r, bufs[r % 2], sems_in[r % 2])

        lane = lax.iota(jnp.int32, _LANES)
        winners = jnp.zeros((_LANES,), jnp.int32)
        neg_inf = jnp.full((_LANES,), -jnp.inf, jnp.float32)
        zero_i = jnp.zeros((_LANES,), jnp.int32)

        for r in range(rows_per):
            in_copies[r].wait()
            buf = bufs[r % 2]

            def step(j, carry):
                ms, tags = carry
                new_ms, new_tags = [], []
                for u in range(_UNROLL):
                    v = buf[j * seg_per_step + u // (_TC // _LANES),
                            pl.ds((u % (_TC // _LANES)) * _LANES, _LANES)]
                    gt = v > ms[u]
                    new_ms.append(jnp.where(gt, v, ms[u]))
                    new_tags.append(jnp.where(gt, j, tags[u]))
                return tuple(new_ms), tuple(new_tags)

            init = ((neg_inf,) * _UNROLL, (zero_i,) * _UNROLL)
            ms, tags = lax.fori_loop(0, steps, step, init)

            pairs = [
                (ms[u], tags[u] * (_UNROLL * _LANES) + (u * _LANES) + lane)
                for u in range(_UNROLL)
            ]
            while len(pairs) > 1:
                nxt = []
                for p in range(0, len(pairs), 2):
                    nxt.append(_merge(*pairs[p], *pairs[p + 1]))
                pairs = nxt
            m, idx = pairs[0]

            for k in (8, 4, 2, 1):
                perm = lane ^ k
                m2 = _xlane_take(m, perm)
                i2 = _xlane_take(idx, perm)
                m, idx = _merge(m, idx, m2, i2)

            if r == 0:
                winners = idx
            else:
                winners = jnp.where(lane == r, idx, winners)

            if r + 2 < rows_per:
                in_copies[r + 2] = start_in(r + 2, bufs[r % 2], sems_in[r % 2])

        cols_v[...] = winners
        pltpu.sync_copy(cols_v, win_hbm.at[wid])

    return argmax_sc


def _onehot_tc_body(rows_blk, cols):
    def body(win_ref, out_ref):
        w = win_ref[0, 0, :]
        col_iota = lax.broadcasted_iota(jnp.int32, (rows_blk, cols), 1)
        out_ref[...] = (col_iota == w[:, None]).astype(jnp.float32)
    return body


def _make_onehot(rows, cols, rows_blk=8):
    return pl.pallas_call(
        _onehot_tc_body(rows_blk, cols),
        grid=(rows // rows_blk,),
        in_specs=[pl.BlockSpec((1, 1, rows_blk), lambda i: (i, 0, 0))],
        out_specs=pl.BlockSpec((rows_blk, cols), lambda i: (i, 0)),
        out_shape=jax.ShapeDtypeStruct((rows, cols), jnp.float32),
    )


def kernel(tensor):
    rows, cols = tensor.shape
    x4 = tensor.reshape(rows // _TR, _TR, cols // _TC, _TC).transpose(0, 2, 1, 3)
    rows_per = rows // (_LANES * 2)
    win2 = _make_argmax(rows, cols)(x4)
    win_rows = win2[:, :rows_per].reshape(rows)
    win_blk = win_rows.reshape(rows // 8, 1, 8)
    return _make_onehot(rows, cols)(win_blk)


# speedup vs baseline: 3.5156x; 1.1752x over previous
import functools

import jax
import jax.numpy as jnp
from jax import lax
from jax.experimental import pallas as pl
from jax.experimental.pallas import tpu as pltpu
from jax.experimental.pallas import tpu_sc as plsc

_LANES = 16
_UNROLL = 8
_TR, _TC = 8, 128


def _xlane_take(x, perm):
    dnums = lax.GatherDimensionNumbers(
        offset_dims=(), collapsed_slice_dims=(0,), start_index_map=(0,))
    return lax.gather(x, perm[:, None], dnums, slice_sizes=(1,),
                      mode=lax.GatherScatterMode.PROMISE_IN_BOUNDS)


def _merge(m_a, i_a, m_b, i_b):
    take_b = (m_b > m_a) | ((m_b == m_a) & (i_b < i_a))
    return jnp.where(take_b, m_b, m_a), jnp.where(take_b, i_b, i_a)


def _make_argmax(rows, cols):
    info = plsc.get_sparse_core_info()
    ncores, nsub = info.num_cores, info.num_subcores
    nworkers = ncores * nsub
    assert rows % nworkers == 0 and rows % _TR == 0 and cols % _TC == 0
    rows_per = rows // nworkers
    assert rows_per % 2 == 0 and rows_per <= _LANES
    assert cols % (_LANES * _UNROLL) == 0
    steps = cols // (_LANES * _UNROLL)
    segs = cols // _TC
    seg_per_step = (_LANES * _UNROLL) // _TC

    mesh = plsc.VectorSubcoreMesh(core_axis_name="c", subcore_axis_name="s")

    @functools.partial(
        pl.kernel,
        out_type=jax.ShapeDtypeStruct((rows, _LANES), jnp.int32),
        mesh=mesh,
        scratch_types=[
            pltpu.VMEM((segs, _TC), jnp.float32),
            pltpu.VMEM((segs, _TC), jnp.float32),
            pltpu.VMEM((rows // nworkers, _LANES), jnp.int32),
            pltpu.SemaphoreType.DMA,
            pltpu.SemaphoreType.DMA,
        ],
    )
    def argmax_sc(x_hbm, win_hbm, in0, in1, cols_v, sem0, sem1):
        wid = lax.axis_index("c") * nsub + lax.axis_index("s")
        row0 = wid * rows_per

        def start_in(rr, buf, sem):
            return pltpu.async_copy(
                x_hbm.at[rr // _TR, :, rr % _TR, :], buf, sem)

        start_in(row0, in0, sem0)
        start_in(row0 + 1, in1, sem1)

        lane = lax.iota(jnp.int32, _LANES)
        neg_inf = jnp.full((_LANES,), -jnp.inf, jnp.float32)
        zero_i = jnp.zeros((_LANES,), jnp.int32)

        def row_argmax(buf):
            def step(j, carry):
                ms, tags = carry
                new_ms, new_tags = [], []
                for u in range(_UNROLL):
                    v = buf[j * seg_per_step + u // (_TC // _LANES),
                            pl.ds((u % (_TC // _LANES)) * _LANES, _LANES)]
                    gt = v > ms[u]
                    new_ms.append(jnp.where(gt, v, ms[u]))
                    new_tags.append(jnp.where(gt, j, tags[u]))
                return tuple(new_ms), tuple(new_tags)

            init = ((neg_inf,) * _UNROLL, (zero_i,) * _UNROLL)
            ms, tags = lax.fori_loop(0, steps, step, init)

            pairs = [
                (ms[u], tags[u] * (_UNROLL * _LANES) + (u * _LANES) + lane)
                for u in range(_UNROLL)
            ]
            while len(pairs) > 1:
                nxt = []
                for p in range(0, len(pairs), 2):
                    nxt.append(_merge(*pairs[p], *pairs[p + 1]))
                pairs = nxt
            m, idx = pairs[0]

            for k in (8, 4, 2, 1):
                perm = lane ^ k
                m2 = _xlane_take(m, perm)
                i2 = _xlane_take(idx, perm)
                m, idx = _merge(m, idx, m2, i2)
            return idx

        npairs = rows_per // 2

        def pair_body(t, carry):
            r_even = row0 + 2 * t
            pltpu.make_async_copy(
                x_hbm.at[r_even // _TR, :, r_even % _TR, :], in0, sem0).wait()
            cols_v[2 * t, :] = row_argmax(in0)

            @pl.when(t < npairs - 1)
            def _():
                start_in(r_even + 2, in0, sem0)

            r_odd = r_even + 1
            pltpu.make_async_copy(
                x_hbm.at[r_odd // _TR, :, r_odd % _TR, :], in1, sem1).wait()
            cols_v[2 * t + 1, :] = row_argmax(in1)

            @pl.when(t < npairs - 1)
            def _():
                start_in(r_odd + 2, in1, sem1)

            return carry

        lax.fori_loop(0, npairs, pair_body, 0)

        pltpu.sync_copy(cols_v, win_hbm.at[pl.ds(row0, rows_per)])

    return argmax_sc


def _onehot_tc_body(rows_blk, cols):
    def body(win_ref, out_ref):
        w = win_ref[:, 0:1]
        col_iota = lax.broadcasted_iota(jnp.int32, (rows_blk, cols), 1)
        out_ref[...] = (col_iota == w).astype(jnp.float32)
    return body


def _make_onehot(rows, cols, rows_blk=32):
    return pl.pallas_call(
        _onehot_tc_body(rows_blk, cols),
        grid=(rows // rows_blk,),
        in_specs=[pl.BlockSpec((rows_blk, _LANES), lambda i: (i, 0))],
        out_specs=pl.BlockSpec((rows_blk, cols), lambda i: (i, 0)),
        out_shape=jax.ShapeDtypeStruct((rows, cols), jnp.float32),
    )


def kernel(tensor):
    rows, cols = tensor.shape
    x4 = tensor.reshape(rows // _TR, _TR, cols // _TC, _TC).transpose(0, 2, 1, 3)
    win2 = _make_argmax(rows, cols)(x4)
    return _make_onehot(rows, cols)(win2)
